# baseline (speedup 1.0000x reference)
import functools

import jax
import jax.numpy as jnp
from jax import lax
from jax.experimental import pallas as pl
from jax.experimental.pallas import tpu as pltpu
from jax.experimental.pallas import tpu_sc as plsc

_NC = 2
_NS = 16
_NW = _NC * _NS

_CHUNK = 128
_NBUF = 3
_DEGW = 128
_BLK = 1000


def _chunk_sizes(total, step):
    sizes = [step] * (total // step)
    if total % step:
        sizes.append(total % step)
    return sizes


def _n_pad(n):
    return -(-n // (_NS * 8)) * (_NS * 8)


def _sc_deg_call(dst, zeros128, n):
    e = dst.shape[0]
    epw = e // _NW
    nchunks = epw // _CHUNK
    assert epw * _NW == e and nchunks * _CHUNK == epw
    np_ = _n_pad(n)
    rows_per_tile = np_ // _NS

    mesh = plsc.VectorSubcoreMesh(core_axis_name="c", subcore_axis_name="s")

    @functools.partial(
        pl.kernel,
        out_type=jax.ShapeDtypeStruct((_NC, np_, _DEGW), jnp.float32),
        mesh=mesh,
        scratch_types=(
            pltpu.VMEM_SHARED((np_, _DEGW), jnp.float32),
            pltpu.VMEM((_CHUNK,), jnp.int32),
            pltpu.VMEM((_CHUNK, _DEGW), jnp.float32),
        ))
    def deg_kernel(dst_hbm, zeros_hbm, degp_hbm, deg_sp, didx_v, ones_v):
        cid = lax.axis_index("c")
        sid = lax.axis_index("s")

        o16 = jnp.ones((16,), jnp.float32)
        lanes = _DEGW // 16

        def fill_ones(i, _):
            ones_v[i // lanes, pl.ds((i % lanes) * 16, 16)] = o16
            return 0
        lax.fori_loop(0, _CHUNK * lanes, fill_ones, 0)

        row0 = sid * rows_per_tile
        off = 0
        for sz in _chunk_sizes(rows_per_tile, zeros128.shape[0]):
            pltpu.sync_copy(zeros_hbm.at[pl.ds(0, sz)],
                            deg_sp.at[pl.ds(row0 + off, sz)])
            off += sz
        plsc.subcore_barrier()

        base = (cid * _NS + sid) * epw

        def step(i, _):
            pltpu.sync_copy(dst_hbm.at[pl.ds(base + i * _CHUNK, _CHUNK)],
                            didx_v)
            pltpu.sync_copy(ones_v, deg_sp.at[didx_v], add=True)
            return 0
        lax.fori_loop(0, nchunks, step, 0)
        plsc.subcore_barrier()

        pltpu.sync_copy(deg_sp.at[pl.ds(row0, rows_per_tile)],
                        degp_hbm.at[cid, pl.ds(row0, rows_per_tile)])

    return deg_kernel(dst, zeros128)


def _sc_agg_call(z, src, dst):
    n, h = z.shape
    e = src.shape[0]
    epw = e // _NW
    nchunks = epw // _CHUNK
    assert epw * _NW == e and nchunks * _CHUNK == epw
    assert nchunks % _NBUF == 0
    np_ = _n_pad(n)
    rows_per_tile = np_ // _NS

    mesh = plsc.VectorSubcoreMesh(core_axis_name="c", subcore_axis_name="s")

    scratch = (
        pltpu.VMEM_SHARED((np_, h), jnp.float32),
        tuple(pltpu.VMEM((_CHUNK,), jnp.int32) for _ in range(_NBUF)),
        tuple(pltpu.VMEM((_CHUNK,), jnp.int32) for _ in range(_NBUF)),
        tuple(pltpu.VMEM((_CHUNK, h), jnp.float32) for _ in range(_NBUF)),
        tuple(pltpu.SemaphoreType.DMA for _ in range(_NBUF)),
        tuple(pltpu.SemaphoreType.DMA for _ in range(_NBUF)),
    )

    @functools.partial(
        pl.kernel,
        out_type=jax.ShapeDtypeStruct((_NC, np_, h), jnp.float32),
        mesh=mesh, scratch_types=scratch)
    def agg_kernel(z_hbm, src_hbm, dst_hbm, out_hbm,
                   agg_sp, sidx, didx, rows, gsems, ssems):
        cid = lax.axis_index("c")
        sid = lax.axis_index("s")

        z16 = jnp.zeros((16,), jnp.float32)
        lanes = h // 16

        def fill_zeros(i, _):
            rows[0][i // lanes, pl.ds((i % lanes) * 16, 16)] = z16
            return 0
        lax.fori_loop(0, _CHUNK * lanes, fill_zeros, 0)

        row0 = sid * rows_per_tile
        off = 0
        for sz in _chunk_sizes(rows_per_tile, _CHUNK):
            pltpu.sync_copy(rows[0].at[pl.ds(0, sz)],
                            agg_sp.at[pl.ds(row0 + off, sz)])
            off += sz
        plsc.subcore_barrier()

        base = (cid * _NS + sid) * epw

        def load_and_gather(i, b):
            off = base + i * _CHUNK
            pltpu.sync_copy(src_hbm.at[pl.ds(off, _CHUNK)], sidx[b])
            pltpu.sync_copy(dst_hbm.at[pl.ds(off, _CHUNK)], didx[b])
            pltpu.async_copy(z_hbm.at[sidx[b]], rows[b], gsems[b])

        def wait_gather(b):
            pltpu.make_async_copy(z_hbm.at[sidx[b]], rows[b],
                                  gsems[b]).wait()

        def wait_scatter(b):
            pltpu.make_async_copy(z_hbm.at[sidx[b]], rows[b],
                                  ssems[b]).wait()

        for b in range(_NBUF - 1):
            load_and_gather(b, b)

        def group(g, _):
            for b in range(_NBUF):
                i = g * _NBUF + b
                bp = (b - 1) % _NBUF
                wait_gather(b)
                pltpu.async_copy(rows[b], agg_sp.at[didx[b]],
                                 ssems[b], add=True)

                @pl.when(i >= 1)
                def _(bp=bp):
                    wait_scatter(bp)

                @pl.when(i + _NBUF - 1 < nchunks)
                def _(i=i, bp=bp):
                    load_and_gather(i + _NBUF - 1, bp)
            return 0
        lax.fori_loop(0, nchunks // _NBUF, group, 0)
        wait_scatter((nchunks - 1) % _NBUF)
        plsc.subcore_barrier()

        pltpu.sync_copy(agg_sp.at[pl.ds(row0, rows_per_tile)],
                        out_hbm.at[cid, pl.ds(row0, rows_per_tile)])

    return agg_kernel(z, src, dst)


def _proj_body(x_ref, wl_ref, wr_ref, bl_ref, z_ref, r_ref):
    xb = x_ref[...]
    z_ref[...] = jnp.dot(xb, wl_ref[...], preferred_element_type=jnp.float32)
    r_ref[...] = (jnp.dot(xb, wr_ref[...], preferred_element_type=jnp.float32)
                  + bl_ref[...])


def _tc_proj(x, wl, wr, bl):
    n, d = x.shape
    h = wl.shape[1]
    return pl.pallas_call(
        _proj_body,
        grid=(n // _BLK,),
        in_specs=[
            pl.BlockSpec((_BLK, d), lambda i: (i, 0)),
            pl.BlockSpec((d, h), lambda i: (0, 0)),
            pl.BlockSpec((d, h), lambda i: (0, 0)),
            pl.BlockSpec((1, h), lambda i: (0, 0)),
        ],
        out_specs=[pl.BlockSpec((_BLK, h), lambda i: (i, 0))] * 2,
        out_shape=[jax.ShapeDtypeStruct((n, h), jnp.float32)] * 2,
    )(x, wl, wr, bl)


def _make_combine_body(emit_out, project):
    def body(s_ref, dp_ref, r_ref, *rest):
        if project:
            wl_ref, wr_ref, bl_ref = rest[:3]
            rest = rest[3:]
        s = s_ref[0] + s_ref[1]
        deg = dp_ref[0] + dp_ref[1]
        inv = 1.0 / jnp.maximum(deg, 1.0)
        out = s * inv + r_ref[...]
        if project:
            hid = jnp.maximum(out, 0.0)
            if emit_out:
                out_ref, g_ref, z_ref, rn_ref = rest
                out_ref[...] = out
                g_ref[...] = hid
            else:
                z_ref, rn_ref = rest
            z_ref[...] = jnp.dot(hid, wl_ref[...],
                                 preferred_element_type=jnp.float32)
            rn_ref[...] = (jnp.dot(hid, wr_ref[...],
                                   preferred_element_type=jnp.float32)
                           + bl_ref[...])
        else:
            (xf_ref,) = rest
            xf_ref[...] = out
    return body


def _tc_combine(s, degp, r, wl=None, wr=None, bl=None, emit_out=False):
    n, h = r.shape
    project = wl is not None
    in_specs = [
        pl.BlockSpec((_NC, _BLK, h), lambda i: (0, i, 0)),
        pl.BlockSpec((_NC, _BLK, _DEGW), lambda i: (0, i, 0)),
        pl.BlockSpec((_BLK, h), lambda i: (i, 0)),
    ]
    args = [s, degp, r]
    n_out = 1
    if project:
        hn = wl.shape[1]
        in_specs += [
            pl.BlockSpec((h, hn), lambda i: (0, 0)),
            pl.BlockSpec((h, hn), lambda i: (0, 0)),
            pl.BlockSpec((1, hn), lambda i: (0, 0)),
        ]
        args += [wl, wr, bl]
        n_out = 4 if emit_out else 2
    outs = pl.pallas_call(
        _make_combine_body(emit_out, project),
        grid=(n // _BLK,),
        in_specs=in_specs,
        out_specs=[pl.BlockSpec((_BLK, h), lambda i: (i, 0))] * n_out,
        out_shape=[jax.ShapeDtypeStruct((n, h), jnp.float32)] * n_out,
    )(*args)
    return outs if n_out > 1 else outs[0]


def kernel(x, edge_index, Wl0, bl0, Wr0, Wl1, bl1, Wr1, Wl2, bl2, Wr2):
    n, d = x.shape
    h = Wl0.shape[1]
    c = Wl2.shape[1]
    e = edge_index.shape[1]

    epw = e // _NW
    assert epw * _NW == e
    step = _NBUF * _CHUNK
    epw_p = -(-epw // step) * step
    assert n < _n_pad(n)
    src = jnp.pad(edge_index[0].reshape(_NW, epw),
                  ((0, 0), (0, epw_p - epw))).reshape(-1)
    dst = jnp.pad(edge_index[1].reshape(_NW, epw),
                  ((0, 0), (0, epw_p - epw)),
                  constant_values=n).reshape(-1)

    bl0r = bl0.reshape(1, h)
    bl1r = bl1.reshape(1, h)
    Wl2p = jnp.pad(Wl2, ((0, 0), (0, h - c)))
    Wr2p = jnp.pad(Wr2, ((0, 0), (0, h - c)))
    bl2p = jnp.pad(bl2, (0, h - c)).reshape(1, h)

    degp = _sc_deg_call(dst, jnp.zeros((128, _DEGW), jnp.float32), n)
    z0, r0 = _tc_proj(x, Wl0, Wr0, bl0r)
    s0 = _sc_agg_call(z0, src, dst)
    z1, r1 = _tc_combine(s0, degp, r0, Wl1, Wr1, bl1r)
    s1 = _sc_agg_call(z1, src, dst)
    out1, g, z2, r2 = _tc_combine(s1, degp, r1, Wl2p, Wr2p, bl2p,
                                  emit_out=True)
    s2 = _sc_agg_call(z2, src, dst)
    xf_pad = _tc_combine(s2, degp, r2)
    return (xf_pad[:, :c], out1, g)


# speedup vs baseline: 2.0315x; 2.0315x over previous
import functools

import jax
import jax.numpy as jnp
from jax import lax
from jax.experimental import pallas as pl
from jax.experimental.pallas import tpu as pltpu
from jax.experimental.pallas import tpu_sc as plsc

_NC = 2
_NS = 16
_NW = _NC * _NS

_CHUNK = 80
_NBUF = 3
_DEGW = 128
_BLK = 1000


def _chunk_sizes(total, step):
    sizes = [step] * (total // step)
    if total % step:
        sizes.append(total % step)
    return sizes


def _n_pad(n):
    return -(-n // (_NS * 8)) * (_NS * 8)


def _sc_deg_call(dst, zeros128, n):
    e = dst.shape[0]
    epw = e // _NW
    nchunks = epw // _CHUNK
    assert epw * _NW == e and nchunks * _CHUNK == epw
    np_ = _n_pad(n)
    rows_per_tile = np_ // _NS

    mesh = plsc.VectorSubcoreMesh(core_axis_name="c", subcore_axis_name="s")

    @functools.partial(
        pl.kernel,
        out_type=jax.ShapeDtypeStruct((_NC, np_, _DEGW), jnp.float32),
        mesh=mesh,
        scratch_types=(
            pltpu.VMEM_SHARED((np_, _DEGW), jnp.float32),
            pltpu.VMEM((_CHUNK,), jnp.int32),
            pltpu.VMEM((_CHUNK, _DEGW), jnp.float32),
        ))
    def deg_kernel(dst_hbm, zeros_hbm, degp_hbm, deg_sp, didx_v, ones_v):
        cid = lax.axis_index("c")
        sid = lax.axis_index("s")

        o16 = jnp.ones((16,), jnp.float32)
        lanes = _DEGW // 16

        def fill_ones(i, _):
            ones_v[i // lanes, pl.ds((i % lanes) * 16, 16)] = o16
            return 0
        lax.fori_loop(0, _CHUNK * lanes, fill_ones, 0)

        row0 = sid * rows_per_tile
        off = 0
        for sz in _chunk_sizes(rows_per_tile, zeros128.shape[0]):
            pltpu.sync_copy(zeros_hbm.at[pl.ds(0, sz)],
                            deg_sp.at[pl.ds(row0 + off, sz)])
            off += sz
        plsc.subcore_barrier()

        base = (cid * _NS + sid) * epw

        def step(i, _):
            pltpu.sync_copy(dst_hbm.at[pl.ds(base + i * _CHUNK, _CHUNK)],
                            didx_v)
            pltpu.sync_copy(ones_v, deg_sp.at[didx_v], add=True)
            return 0
        lax.fori_loop(0, nchunks, step, 0)
        plsc.subcore_barrier()

        pltpu.sync_copy(deg_sp.at[pl.ds(row0, rows_per_tile)],
                        degp_hbm.at[cid, pl.ds(row0, rows_per_tile)])

    return deg_kernel(dst, zeros128)


def _sc_agg_call(z, src, dst):
    n, h = z.shape
    e = src.shape[0]
    epw = e // _NW
    nchunks = epw // _CHUNK
    assert epw * _NW == e and nchunks * _CHUNK == epw
    assert nchunks % _NBUF == 0
    np_ = _n_pad(n)
    rows_per_tile = np_ // _NS

    mesh = plsc.VectorSubcoreMesh(core_axis_name="c", subcore_axis_name="s")

    scratch = (
        pltpu.VMEM_SHARED((np_, h), jnp.float32),
        tuple(pltpu.VMEM((_CHUNK,), jnp.int32) for _ in range(_NBUF)),
        tuple(pltpu.VMEM((_CHUNK,), jnp.int32) for _ in range(_NBUF)),
        tuple(pltpu.VMEM((_CHUNK, h), jnp.float32) for _ in range(_NBUF)),
        tuple(pltpu.SemaphoreType.DMA for _ in range(_NBUF)),
        tuple(pltpu.SemaphoreType.DMA for _ in range(_NBUF)),
    )

    @functools.partial(
        pl.kernel,
        out_type=jax.ShapeDtypeStruct((_NC, np_, h), jnp.float32),
        mesh=mesh, scratch_types=scratch)
    def agg_kernel(z_hbm, src_hbm, dst_hbm, out_hbm,
                   agg_sp, sidx, didx, rows, gsems, ssems):
        cid = lax.axis_index("c")
        sid = lax.axis_index("s")

        z16 = jnp.zeros((16,), jnp.float32)
        lanes = h // 16

        def fill_zeros(i, _):
            rows[0][i // lanes, pl.ds((i % lanes) * 16, 16)] = z16
            return 0
        lax.fori_loop(0, _CHUNK * lanes, fill_zeros, 0)

        row0 = sid * rows_per_tile
        off = 0
        for sz in _chunk_sizes(rows_per_tile, _CHUNK):
            pltpu.sync_copy(rows[0].at[pl.ds(0, sz)],
                            agg_sp.at[pl.ds(row0 + off, sz)])
            off += sz
        plsc.subcore_barrier()

        base = (cid * _NS + sid) * epw

        def load_and_gather(i, b):
            off = base + i * _CHUNK
            pltpu.sync_copy(src_hbm.at[pl.ds(off, _CHUNK)], sidx[b])
            pltpu.sync_copy(dst_hbm.at[pl.ds(off, _CHUNK)], didx[b])
            pltpu.async_copy(z_hbm.at[sidx[b]], rows[b], gsems[b])

        def wait_gather(b):
            pltpu.make_async_copy(z_hbm.at[sidx[b]], rows[b],
                                  gsems[b]).wait()

        def wait_scatter(b):
            pltpu.make_async_copy(z_hbm.at[sidx[b]], rows[b],
                                  ssems[b]).wait()

        for b in range(_NBUF - 1):
            load_and_gather(b, b)

        def group(g, _):
            for b in range(_NBUF):
                i = g * _NBUF + b
                bp = (b - 1) % _NBUF
                wait_gather(b)
                pltpu.async_copy(rows[b], agg_sp.at[didx[b]],
                                 ssems[b], add=True)

                @pl.when(i >= 1)
                def _(bp=bp):
                    wait_scatter(bp)

                @pl.when(i + _NBUF - 1 < nchunks)
                def _(i=i, bp=bp):
                    load_and_gather(i + _NBUF - 1, bp)
            return 0
        lax.fori_loop(0, nchunks // _NBUF, group, 0)
        wait_scatter((nchunks - 1) % _NBUF)
        plsc.subcore_barrier()

        pltpu.sync_copy(agg_sp.at[pl.ds(row0, rows_per_tile)],
                        out_hbm.at[cid, pl.ds(row0, rows_per_tile)])

    return agg_kernel(z, src, dst)


def _proj_body(x_ref, wl_ref, wr_ref, bl_ref, z_ref, r_ref):
    xb = x_ref[...]
    z_ref[...] = jnp.dot(xb, wl_ref[...], preferred_element_type=jnp.float32)
    r_ref[...] = (jnp.dot(xb, wr_ref[...], preferred_element_type=jnp.float32)
                  + bl_ref[...])


def _tc_proj(x, wl, wr, bl):
    n, d = x.shape
    h = wl.shape[1]
    return pl.pallas_call(
        _proj_body,
        grid=(n // _BLK,),
        in_specs=[
            pl.BlockSpec((_BLK, d), lambda i: (i, 0)),
            pl.BlockSpec((d, h), lambda i: (0, 0)),
            pl.BlockSpec((d, h), lambda i: (0, 0)),
            pl.BlockSpec((1, h), lambda i: (0, 0)),
        ],
        out_specs=[pl.BlockSpec((_BLK, h), lambda i: (i, 0))] * 2,
        out_shape=[jax.ShapeDtypeStruct((n, h), jnp.float32)] * 2,
    )(x, wl, wr, bl)


def _make_combine_body(emit_out, project):
    def body(s_ref, dp_ref, r_ref, *rest):
        if project:
            wl_ref, wr_ref, bl_ref = rest[:3]
            rest = rest[3:]
        s = s_ref[0] + s_ref[1]
        deg = dp_ref[0] + dp_ref[1]
        inv = 1.0 / jnp.maximum(deg, 1.0)
        out = s * inv + r_ref[...]
        if project:
            hid = jnp.maximum(out, 0.0)
            if emit_out:
                out_ref, g_ref, z_ref, rn_ref = rest
                out_ref[...] = out
                g_ref[...] = hid
            else:
                z_ref, rn_ref = rest
            z_ref[...] = jnp.dot(hid, wl_ref[...],
                                 preferred_element_type=jnp.float32)
            rn_ref[...] = (jnp.dot(hid, wr_ref[...],
                                   preferred_element_type=jnp.float32)
                           + bl_ref[...])
        else:
            (xf_ref,) = rest
            xf_ref[...] = out
    return body


def _tc_combine(s, degp, r, wl=None, wr=None, bl=None, emit_out=False):
    n, h = r.shape
    project = wl is not None
    in_specs = [
        pl.BlockSpec((_NC, _BLK, h), lambda i: (0, i, 0)),
        pl.BlockSpec((_NC, _BLK, _DEGW), lambda i: (0, i, 0)),
        pl.BlockSpec((_BLK, h), lambda i: (i, 0)),
    ]
    args = [s, degp, r]
    n_out = 1
    if project:
        hn = wl.shape[1]
        in_specs += [
            pl.BlockSpec((h, hn), lambda i: (0, 0)),
            pl.BlockSpec((h, hn), lambda i: (0, 0)),
            pl.BlockSpec((1, hn), lambda i: (0, 0)),
        ]
        args += [wl, wr, bl]
        n_out = 4 if emit_out else 2
    outs = pl.pallas_call(
        _make_combine_body(emit_out, project),
        grid=(n // _BLK,),
        in_specs=in_specs,
        out_specs=[pl.BlockSpec((_BLK, h), lambda i: (i, 0))] * n_out,
        out_shape=[jax.ShapeDtypeStruct((n, h), jnp.float32)] * n_out,
    )(*args)
    return outs if n_out > 1 else outs[0]


def kernel(x, edge_index, Wl0, bl0, Wr0, Wl1, bl1, Wr1, Wl2, bl2, Wr2):
    n, d = x.shape
    h = Wl0.shape[1]
    c = Wl2.shape[1]
    e = edge_index.shape[1]

    epw = e // _NW
    assert epw * _NW == e
    step = _NBUF * _CHUNK
    epw_p = -(-epw // step) * step
    assert n < _n_pad(n)
    src = jnp.pad(edge_index[0].reshape(_NW, epw),
                  ((0, 0), (0, epw_p - epw))).reshape(-1)
    dst = jnp.pad(edge_index[1].reshape(_NW, epw),
                  ((0, 0), (0, epw_p - epw)),
                  constant_values=n).reshape(-1)

    bl0r = bl0.reshape(1, h)
    bl1r = bl1.reshape(1, h)
    Wl2p = jnp.pad(Wl2, ((0, 0), (0, h - c)))
    Wr2p = jnp.pad(Wr2, ((0, 0), (0, h - c)))
    bl2p = jnp.pad(bl2, (0, h - c)).reshape(1, h)

    degp = _sc_deg_call(dst, jnp.zeros((128, _DEGW), jnp.float32), n)
    z0, r0 = _tc_proj(x, Wl0, Wr0, bl0r)
    s0 = _sc_agg_call(z0, src, dst)
    z1, r1 = _tc_combine(s0, degp, r0, Wl1, Wr1, bl1r)
    s1 = _sc_agg_call(z1, src, dst)
    out1, g, z2, r2 = _tc_combine(s1, degp, r1, Wl2p, Wr2p, bl2p,
                                  emit_out=True)
    s2 = _sc_agg_call(z2, src, dst)
    xf_pad = _tc_combine(s2, degp, r2)
    return (xf_pad[:, :c], out1, g)


# speedup vs baseline: 2.1997x; 1.0828x over previous
import functools

import jax
import jax.numpy as jnp
from jax import lax
from jax.experimental import pallas as pl
from jax.experimental.pallas import tpu as pltpu
from jax.experimental.pallas import tpu_sc as plsc

_NC = 2
_NS = 16
_NW = _NC * _NS

_CHUNK = 112
_NBUF = 3
_DEGW = 128
_BLK = 1000


def _chunk_sizes(total, step):
    sizes = [step] * (total // step)
    if total % step:
        sizes.append(total % step)
    return sizes


def _n_pad(n):
    return -(-n // (_NS * 8)) * (_NS * 8)


def _sc_deg_call(dst, zeros128, n):
    e = dst.shape[0]
    epw = e // _NW
    nchunks = epw // _CHUNK
    assert epw * _NW == e and nchunks * _CHUNK == epw
    np_ = _n_pad(n)
    rows_per_tile = np_ // _NS

    mesh = plsc.VectorSubcoreMesh(core_axis_name="c", subcore_axis_name="s")

    @functools.partial(
        pl.kernel,
        out_type=jax.ShapeDtypeStruct((_NC, np_, _DEGW), jnp.float32),
        mesh=mesh,
        scratch_types=(
            pltpu.VMEM_SHARED((np_, _DEGW), jnp.float32),
            pltpu.VMEM((_CHUNK,), jnp.int32),
            pltpu.VMEM((_CHUNK, _DEGW), jnp.float32),
        ))
    def deg_kernel(dst_hbm, zeros_hbm, degp_hbm, deg_sp, didx_v, ones_v):
        cid = lax.axis_index("c")
        sid = lax.axis_index("s")

        o16 = jnp.ones((16,), jnp.float32)
        lanes = _DEGW // 16

        def fill_ones(i, _):
            ones_v[i // lanes, pl.ds((i % lanes) * 16, 16)] = o16
            return 0
        lax.fori_loop(0, _CHUNK * lanes, fill_ones, 0)

        row0 = sid * rows_per_tile
        off = 0
        for sz in _chunk_sizes(rows_per_tile, zeros128.shape[0]):
            pltpu.sync_copy(zeros_hbm.at[pl.ds(0, sz)],
                            deg_sp.at[pl.ds(row0 + off, sz)])
            off += sz
        plsc.subcore_barrier()

        base = (cid * _NS + sid) * epw

        def step(i, _):
            pltpu.sync_copy(dst_hbm.at[pl.ds(base + i * _CHUNK, _CHUNK)],
                            didx_v)
            pltpu.sync_copy(ones_v, deg_sp.at[didx_v], add=True)
            return 0
        lax.fori_loop(0, nchunks, step, 0)
        plsc.subcore_barrier()

        pltpu.sync_copy(deg_sp.at[pl.ds(row0, rows_per_tile)],
                        degp_hbm.at[cid, pl.ds(row0, rows_per_tile)])

    return deg_kernel(dst, zeros128)


def _sc_agg_call(z, src, dst):
    n, h = z.shape
    e = src.shape[0]
    epw = e // _NW
    nchunks = epw // _CHUNK
    assert epw * _NW == e and nchunks * _CHUNK == epw
    assert nchunks % _NBUF == 0
    np_ = _n_pad(n)
    rows_per_tile = np_ // _NS

    mesh = plsc.VectorSubcoreMesh(core_axis_name="c", subcore_axis_name="s")

    scratch = (
        pltpu.VMEM_SHARED((np_, h), jnp.float32),
        tuple(pltpu.VMEM((_CHUNK,), jnp.int32) for _ in range(_NBUF)),
        tuple(pltpu.VMEM((_CHUNK,), jnp.int32) for _ in range(_NBUF)),
        tuple(pltpu.VMEM((_CHUNK, h), jnp.float32) for _ in range(_NBUF)),
        tuple(pltpu.SemaphoreType.DMA for _ in range(_NBUF)),
        tuple(pltpu.SemaphoreType.DMA for _ in range(_NBUF)),
    )

    @functools.partial(
        pl.kernel,
        out_type=jax.ShapeDtypeStruct((_NC, np_, h), jnp.float32),
        mesh=mesh, scratch_types=scratch)
    def agg_kernel(z_hbm, src_hbm, dst_hbm, out_hbm,
                   agg_sp, sidx, didx, rows, gsems, ssems):
        cid = lax.axis_index("c")
        sid = lax.axis_index("s")

        z16 = jnp.zeros((16,), jnp.float32)
        lanes = h // 16

        def fill_zeros(i, _):
            rows[0][i // lanes, pl.ds((i % lanes) * 16, 16)] = z16
            return 0
        lax.fori_loop(0, _CHUNK * lanes, fill_zeros, 0)

        row0 = sid * rows_per_tile
        off = 0
        for sz in _chunk_sizes(rows_per_tile, _CHUNK):
            pltpu.sync_copy(rows[0].at[pl.ds(0, sz)],
                            agg_sp.at[pl.ds(row0 + off, sz)])
            off += sz
        plsc.subcore_barrier()

        base = (cid * _NS + sid) * epw

        def load_and_gather(i, b):
            off = base + i * _CHUNK
            pltpu.sync_copy(src_hbm.at[pl.ds(off, _CHUNK)], sidx[b])
            pltpu.sync_copy(dst_hbm.at[pl.ds(off, _CHUNK)], didx[b])
            pltpu.async_copy(z_hbm.at[sidx[b]], rows[b], gsems[b])

        def wait_gather(b):
            pltpu.make_async_copy(z_hbm.at[sidx[b]], rows[b],
                                  gsems[b]).wait()

        def wait_scatter(b):
            pltpu.make_async_copy(z_hbm.at[sidx[b]], rows[b],
                                  ssems[b]).wait()

        for b in range(_NBUF - 1):
            load_and_gather(b, b)

        def group(g, _):
            for b in range(_NBUF):
                i = g * _NBUF + b
                bp = (b - 1) % _NBUF
                wait_gather(b)
                pltpu.async_copy(rows[b], agg_sp.at[didx[b]],
                                 ssems[b], add=True)

                @pl.when(i >= 1)
                def _(bp=bp):
                    wait_scatter(bp)

                @pl.when(i + _NBUF - 1 < nchunks)
                def _(i=i, bp=bp):
                    load_and_gather(i + _NBUF - 1, bp)
            return 0
        lax.fori_loop(0, nchunks // _NBUF, group, 0)
        wait_scatter((nchunks - 1) % _NBUF)
        plsc.subcore_barrier()

        pltpu.sync_copy(agg_sp.at[pl.ds(row0, rows_per_tile)],
                        out_hbm.at[cid, pl.ds(row0, rows_per_tile)])

    return agg_kernel(z, src, dst)


def _proj_body(x_ref, wl_ref, wr_ref, bl_ref, z_ref, r_ref):
    xb = x_ref[...]
    z_ref[...] = jnp.dot(xb, wl_ref[...], preferred_element_type=jnp.float32)
    r_ref[...] = (jnp.dot(xb, wr_ref[...], preferred_element_type=jnp.float32)
                  + bl_ref[...])


def _tc_proj(x, wl, wr, bl):
    n, d = x.shape
    h = wl.shape[1]
    return pl.pallas_call(
        _proj_body,
        grid=(n // _BLK,),
        in_specs=[
            pl.BlockSpec((_BLK, d), lambda i: (i, 0)),
            pl.BlockSpec((d, h), lambda i: (0, 0)),
            pl.BlockSpec((d, h), lambda i: (0, 0)),
            pl.BlockSpec((1, h), lambda i: (0, 0)),
        ],
        out_specs=[pl.BlockSpec((_BLK, h), lambda i: (i, 0))] * 2,
        out_shape=[jax.ShapeDtypeStruct((n, h), jnp.float32)] * 2,
    )(x, wl, wr, bl)


def _make_combine_body(emit_out, project):
    def body(s_ref, dp_ref, r_ref, *rest):
        if project:
            wl_ref, wr_ref, bl_ref = rest[:3]
            rest = rest[3:]
        s = s_ref[0] + s_ref[1]
        deg = dp_ref[0] + dp_ref[1]
        inv = 1.0 / jnp.maximum(deg, 1.0)
        out = s * inv + r_ref[...]
        if project:
            hid = jnp.maximum(out, 0.0)
            if emit_out:
                out_ref, g_ref, z_ref, rn_ref = rest
                out_ref[...] = out
                g_ref[...] = hid
            else:
                z_ref, rn_ref = rest
            z_ref[...] = jnp.dot(hid, wl_ref[...],
                                 preferred_element_type=jnp.float32)
            rn_ref[...] = (jnp.dot(hid, wr_ref[...],
                                   preferred_element_type=jnp.float32)
                           + bl_ref[...])
        else:
            (xf_ref,) = rest
            xf_ref[...] = out
    return body


def _tc_combine(s, degp, r, wl=None, wr=None, bl=None, emit_out=False):
    n, h = r.shape
    project = wl is not None
    in_specs = [
        pl.BlockSpec((_NC, _BLK, h), lambda i: (0, i, 0)),
        pl.BlockSpec((_NC, _BLK, _DEGW), lambda i: (0, i, 0)),
        pl.BlockSpec((_BLK, h), lambda i: (i, 0)),
    ]
    args = [s, degp, r]
    n_out = 1
    if project:
        hn = wl.shape[1]
        in_specs += [
            pl.BlockSpec((h, hn), lambda i: (0, 0)),
            pl.BlockSpec((h, hn), lambda i: (0, 0)),
            pl.BlockSpec((1, hn), lambda i: (0, 0)),
        ]
        args += [wl, wr, bl]
        n_out = 4 if emit_out else 2
    outs = pl.pallas_call(
        _make_combine_body(emit_out, project),
        grid=(n // _BLK,),
        in_specs=in_specs,
        out_specs=[pl.BlockSpec((_BLK, h), lambda i: (i, 0))] * n_out,
        out_shape=[jax.ShapeDtypeStruct((n, h), jnp.float32)] * n_out,
    )(*args)
    return outs if n_out > 1 else outs[0]


def kernel(x, edge_index, Wl0, bl0, Wr0, Wl1, bl1, Wr1, Wl2, bl2, Wr2):
    n, d = x.shape
    h = Wl0.shape[1]
    c = Wl2.shape[1]
    e = edge_index.shape[1]

    epw = e // _NW
    assert epw * _NW == e
    step = _NBUF * _CHUNK
    epw_p = -(-epw // step) * step
    assert n < _n_pad(n)
    src = jnp.pad(edge_index[0].reshape(_NW, epw),
                  ((0, 0), (0, epw_p - epw))).reshape(-1)
    dst = jnp.pad(edge_index[1].reshape(_NW, epw),
                  ((0, 0), (0, epw_p - epw)),
                  constant_values=n).reshape(-1)

    bl0r = bl0.reshape(1, h)
    bl1r = bl1.reshape(1, h)
    Wl2p = jnp.pad(Wl2, ((0, 0), (0, h - c)))
    Wr2p = jnp.pad(Wr2, ((0, 0), (0, h - c)))
    bl2p = jnp.pad(bl2, (0, h - c)).reshape(1, h)

    degp = _sc_deg_call(dst, jnp.zeros((128, _DEGW), jnp.float32), n)
    z0, r0 = _tc_proj(x, Wl0, Wr0, bl0r)
    s0 = _sc_agg_call(z0, src, dst)
    z1, r1 = _tc_combine(s0, degp, r0, Wl1, Wr1, bl1r)
    s1 = _sc_agg_call(z1, src, dst)
    out1, g, z2, r2 = _tc_combine(s1, degp, r1, Wl2p, Wr2p, bl2p,
                                  emit_out=True)
    s2 = _sc_agg_call(z2, src, dst)
    xf_pad = _tc_combine(s2, degp, r2)
    return (xf_pad[:, :c], out1, g)


# speedup vs baseline: 2.2244x; 1.0112x over previous
import functools

import jax
import jax.numpy as jnp
from jax import lax
from jax.experimental import pallas as pl
from jax.experimental.pallas import tpu as pltpu
from jax.experimental.pallas import tpu_sc as plsc

_NC = 2
_NS = 16
_NW = _NC * _NS

_CHUNK = 120
_NBUF = 3
_DEGW = 128
_BLK = 1000


def _chunk_sizes(total, step):
    sizes = [step] * (total // step)
    if total % step:
        sizes.append(total % step)
    return sizes


def _n_pad(n):
    return -(-n // (_NS * 8)) * (_NS * 8)


def _sc_deg_call(dst, zeros128, n):
    e = dst.shape[0]
    epw = e // _NW
    nchunks = epw // _CHUNK
    assert epw * _NW == e and nchunks * _CHUNK == epw
    np_ = _n_pad(n)
    rows_per_tile = np_ // _NS

    mesh = plsc.VectorSubcoreMesh(core_axis_name="c", subcore_axis_name="s")

    @functools.partial(
        pl.kernel,
        out_type=jax.ShapeDtypeStruct((_NC, np_, _DEGW), jnp.float32),
        mesh=mesh,
        scratch_types=(
            pltpu.VMEM_SHARED((np_, _DEGW), jnp.float32),
            pltpu.VMEM((_CHUNK,), jnp.int32),
            pltpu.VMEM((_CHUNK, _DEGW), jnp.float32),
        ))
    def deg_kernel(dst_hbm, zeros_hbm, degp_hbm, deg_sp, didx_v, ones_v):
        cid = lax.axis_index("c")
        sid = lax.axis_index("s")

        o16 = jnp.ones((16,), jnp.float32)
        lanes = _DEGW // 16

        def fill_ones(i, _):
            ones_v[i // lanes, pl.ds((i % lanes) * 16, 16)] = o16
            return 0
        lax.fori_loop(0, _CHUNK * lanes, fill_ones, 0)

        row0 = sid * rows_per_tile
        off = 0
        for sz in _chunk_sizes(rows_per_tile, zeros128.shape[0]):
            pltpu.sync_copy(zeros_hbm.at[pl.ds(0, sz)],
                            deg_sp.at[pl.ds(row0 + off, sz)])
            off += sz
        plsc.subcore_barrier()

        base = (cid * _NS + sid) * epw

        def step(i, _):
            pltpu.sync_copy(dst_hbm.at[pl.ds(base + i * _CHUNK, _CHUNK)],
                            didx_v)
            pltpu.sync_copy(ones_v, deg_sp.at[didx_v], add=True)
            return 0
        lax.fori_loop(0, nchunks, step, 0)
        plsc.subcore_barrier()

        pltpu.sync_copy(deg_sp.at[pl.ds(row0, rows_per_tile)],
                        degp_hbm.at[cid, pl.ds(row0, rows_per_tile)])

    return deg_kernel(dst, zeros128)


def _sc_agg_call(z, src, dst):
    n, h = z.shape
    e = src.shape[0]
    epw = e // _NW
    nchunks = epw // _CHUNK
    assert epw * _NW == e and nchunks * _CHUNK == epw
    assert nchunks % _NBUF == 0
    np_ = _n_pad(n)
    rows_per_tile = np_ // _NS

    mesh = plsc.VectorSubcoreMesh(core_axis_name="c", subcore_axis_name="s")

    scratch = (
        pltpu.VMEM_SHARED((np_, h), jnp.float32),
        tuple(pltpu.VMEM((_CHUNK,), jnp.int32) for _ in range(_NBUF)),
        tuple(pltpu.VMEM((_CHUNK,), jnp.int32) for _ in range(_NBUF)),
        tuple(pltpu.VMEM((_CHUNK, h), jnp.float32) for _ in range(_NBUF)),
        tuple(pltpu.SemaphoreType.DMA for _ in range(_NBUF)),
        tuple(pltpu.SemaphoreType.DMA for _ in range(_NBUF)),
    )

    @functools.partial(
        pl.kernel,
        out_type=jax.ShapeDtypeStruct((_NC, np_, h), jnp.float32),
        mesh=mesh, scratch_types=scratch)
    def agg_kernel(z_hbm, src_hbm, dst_hbm, out_hbm,
                   agg_sp, sidx, didx, rows, gsems, ssems):
        cid = lax.axis_index("c")
        sid = lax.axis_index("s")

        z16 = jnp.zeros((16,), jnp.float32)
        lanes = h // 16

        def fill_zeros(i, _):
            rows[0][i // lanes, pl.ds((i % lanes) * 16, 16)] = z16
            return 0
        lax.fori_loop(0, _CHUNK * lanes, fill_zeros, 0)

        row0 = sid * rows_per_tile
        off = 0
        for sz in _chunk_sizes(rows_per_tile, _CHUNK):
            pltpu.sync_copy(rows[0].at[pl.ds(0, sz)],
                            agg_sp.at[pl.ds(row0 + off, sz)])
            off += sz
        plsc.subcore_barrier()

        base = (cid * _NS + sid) * epw

        def load_and_gather(i, b):
            off = base + i * _CHUNK
            pltpu.sync_copy(src_hbm.at[pl.ds(off, _CHUNK)], sidx[b])
            pltpu.sync_copy(dst_hbm.at[pl.ds(off, _CHUNK)], didx[b])
            pltpu.async_copy(z_hbm.at[sidx[b]], rows[b], gsems[b])

        def wait_gather(b):
            pltpu.make_async_copy(z_hbm.at[sidx[b]], rows[b],
                                  gsems[b]).wait()

        def wait_scatter(b):
            pltpu.make_async_copy(z_hbm.at[sidx[b]], rows[b],
                                  ssems[b]).wait()

        for b in range(_NBUF - 1):
            load_and_gather(b, b)

        def group(g, _):
            for b in range(_NBUF):
                i = g * _NBUF + b
                bp = (b - 1) % _NBUF
                wait_gather(b)
                pltpu.async_copy(rows[b], agg_sp.at[didx[b]],
                                 ssems[b], add=True)

                @pl.when(i >= 1)
                def _(bp=bp):
                    wait_scatter(bp)

                @pl.when(i + _NBUF - 1 < nchunks)
                def _(i=i, bp=bp):
                    load_and_gather(i + _NBUF - 1, bp)
            return 0
        lax.fori_loop(0, nchunks // _NBUF, group, 0)
        wait_scatter((nchunks - 1) % _NBUF)
        plsc.subcore_barrier()

        pltpu.sync_copy(agg_sp.at[pl.ds(row0, rows_per_tile)],
                        out_hbm.at[cid, pl.ds(row0, rows_per_tile)])

    return agg_kernel(z, src, dst)


def _proj_body(x_ref, wl_ref, wr_ref, bl_ref, z_ref, r_ref):
    xb = x_ref[...]
    z_ref[...] = jnp.dot(xb, wl_ref[...], preferred_element_type=jnp.float32)
    r_ref[...] = (jnp.dot(xb, wr_ref[...], preferred_element_type=jnp.float32)
                  + bl_ref[...])


def _tc_proj(x, wl, wr, bl):
    n, d = x.shape
    h = wl.shape[1]
    return pl.pallas_call(
        _proj_body,
        grid=(n // _BLK,),
        in_specs=[
            pl.BlockSpec((_BLK, d), lambda i: (i, 0)),
            pl.BlockSpec((d, h), lambda i: (0, 0)),
            pl.BlockSpec((d, h), lambda i: (0, 0)),
            pl.BlockSpec((1, h), lambda i: (0, 0)),
        ],
        out_specs=[pl.BlockSpec((_BLK, h), lambda i: (i, 0))] * 2,
        out_shape=[jax.ShapeDtypeStruct((n, h), jnp.float32)] * 2,
    )(x, wl, wr, bl)


def _make_combine_body(emit_out, project):
    def body(s_ref, dp_ref, r_ref, *rest):
        if project:
            wl_ref, wr_ref, bl_ref = rest[:3]
            rest = rest[3:]
        s = s_ref[0] + s_ref[1]
        deg = dp_ref[0] + dp_ref[1]
        inv = 1.0 / jnp.maximum(deg, 1.0)
        out = s * inv + r_ref[...]
        if project:
            hid = jnp.maximum(out, 0.0)
            if emit_out:
                out_ref, g_ref, z_ref, rn_ref = rest
                out_ref[...] = out
                g_ref[...] = hid
            else:
                z_ref, rn_ref = rest
            z_ref[...] = jnp.dot(hid, wl_ref[...],
                                 preferred_element_type=jnp.float32)
            rn_ref[...] = (jnp.dot(hid, wr_ref[...],
                                   preferred_element_type=jnp.float32)
                           + bl_ref[...])
        else:
            (xf_ref,) = rest
            xf_ref[...] = out
    return body


def _tc_combine(s, degp, r, wl=None, wr=None, bl=None, emit_out=False):
    n, h = r.shape
    project = wl is not None
    in_specs = [
        pl.BlockSpec((_NC, _BLK, h), lambda i: (0, i, 0)),
        pl.BlockSpec((_NC, _BLK, _DEGW), lambda i: (0, i, 0)),
        pl.BlockSpec((_BLK, h), lambda i: (i, 0)),
    ]
    args = [s, degp, r]
    n_out = 1
    if project:
        hn = wl.shape[1]
        in_specs += [
            pl.BlockSpec((h, hn), lambda i: (0, 0)),
            pl.BlockSpec((h, hn), lambda i: (0, 0)),
            pl.BlockSpec((1, hn), lambda i: (0, 0)),
        ]
        args += [wl, wr, bl]
        n_out = 4 if emit_out else 2
    outs = pl.pallas_call(
        _make_combine_body(emit_out, project),
        grid=(n // _BLK,),
        in_specs=in_specs,
        out_specs=[pl.BlockSpec((_BLK, h), lambda i: (i, 0))] * n_out,
        out_shape=[jax.ShapeDtypeStruct((n, h), jnp.float32)] * n_out,
    )(*args)
    return outs if n_out > 1 else outs[0]


def kernel(x, edge_index, Wl0, bl0, Wr0, Wl1, bl1, Wr1, Wl2, bl2, Wr2):
    n, d = x.shape
    h = Wl0.shape[1]
    c = Wl2.shape[1]
    e = edge_index.shape[1]

    epw = e // _NW
    assert epw * _NW == e
    step = _NBUF * _CHUNK
    epw_p = -(-epw // step) * step
    assert n < _n_pad(n)
    src = jnp.pad(edge_index[0].reshape(_NW, epw),
                  ((0, 0), (0, epw_p - epw))).reshape(-1)
    dst = jnp.pad(edge_index[1].reshape(_NW, epw),
                  ((0, 0), (0, epw_p - epw)),
                  constant_values=n).reshape(-1)

    bl0r = bl0.reshape(1, h)
    bl1r = bl1.reshape(1, h)
    Wl2p = jnp.pad(Wl2, ((0, 0), (0, h - c)))
    Wr2p = jnp.pad(Wr2, ((0, 0), (0, h - c)))
    bl2p = jnp.pad(bl2, (0, h - c)).reshape(1, h)

    degp = _sc_deg_call(dst, jnp.zeros((128, _DEGW), jnp.float32), n)
    z0, r0 = _tc_proj(x, Wl0, Wr0, bl0r)
    s0 = _sc_agg_call(z0, src, dst)
    z1, r1 = _tc_combine(s0, degp, r0, Wl1, Wr1, bl1r)
    s1 = _sc_agg_call(z1, src, dst)
    out1, g, z2, r2 = _tc_combine(s1, degp, r1, Wl2p, Wr2p, bl2p,
                                  emit_out=True)
    s2 = _sc_agg_call(z2, src, dst)
    xf_pad = _tc_combine(s2, degp, r2)
    return (xf_pad[:, :c], out1, g)


# speedup vs baseline: 2.3106x; 1.0388x over previous
import functools

import jax
import jax.numpy as jnp
from jax import lax
from jax.experimental import pallas as pl
from jax.experimental.pallas import tpu as pltpu
from jax.experimental.pallas import tpu_sc as plsc

_NC = 2
_NS = 16
_NW = _NC * _NS

_CHUNK = 120
_NBUF = 3
_DEGW = 128
_BLK = 1000


def _chunk_sizes(total, step):
    sizes = [step] * (total // step)
    if total % step:
        sizes.append(total % step)
    return sizes


def _n_pad(n):
    return -(-n // (_NS * 8)) * (_NS * 8)


def _sc_deg_call(dst, zeros128, n):
    e = dst.shape[0]
    epw = e // _NW
    nchunks = epw // _CHUNK
    assert epw * _NW == e and nchunks * _CHUNK == epw
    np_ = _n_pad(n)
    rows_per_tile = np_ // _NS

    mesh = plsc.VectorSubcoreMesh(core_axis_name="c", subcore_axis_name="s")

    @functools.partial(
        pl.kernel,
        out_type=jax.ShapeDtypeStruct((_NC, np_, _DEGW), jnp.float32),
        mesh=mesh,
        scratch_types=(
            pltpu.VMEM_SHARED((np_, _DEGW), jnp.float32),
            tuple(pltpu.VMEM((_CHUNK,), jnp.int32) for _ in range(2)),
            pltpu.VMEM((_CHUNK, _DEGW), jnp.float32),
            tuple(pltpu.SemaphoreType.DMA for _ in range(2)),
            tuple(pltpu.SemaphoreType.DMA for _ in range(2)),
        ))
    def deg_kernel(dst_hbm, zeros_hbm, degp_hbm, deg_sp, didx, ones_v,
                   isems, ssems):
        cid = lax.axis_index("c")
        sid = lax.axis_index("s")

        o16 = jnp.ones((16,), jnp.float32)
        lanes = _DEGW // 16

        def fill_ones(i, _):
            ones_v[i // lanes, pl.ds((i % lanes) * 16, 16)] = o16
            return 0
        lax.fori_loop(0, _CHUNK * lanes, fill_ones, 0)

        row0 = sid * rows_per_tile
        off = 0
        for sz in _chunk_sizes(rows_per_tile, zeros128.shape[0]):
            pltpu.sync_copy(zeros_hbm.at[pl.ds(0, sz)],
                            deg_sp.at[pl.ds(row0 + off, sz)])
            off += sz
        plsc.subcore_barrier()

        base = (cid * _NS + sid) * epw

        def load_idx(i, b):
            pltpu.async_copy(dst_hbm.at[pl.ds(base + i * _CHUNK, _CHUNK)],
                             didx[b], isems[b])

        def wait_idx(b):
            pltpu.make_async_copy(dst_hbm.at[pl.ds(base, _CHUNK)],
                                  didx[b], isems[b]).wait()

        def wait_scatter(b):
            pltpu.make_async_copy(ones_v, deg_sp.at[didx[b]],
                                  ssems[b]).wait()

        load_idx(0, 0)

        def group(g, _):
            for b in range(2):
                i = g * 2 + b
                wait_idx(b)

                @pl.when(i >= 1)
                def _(b=b):
                    wait_scatter(1 - b)

                @pl.when(i + 1 < nchunks)
                def _(i=i, b=b):
                    load_idx(i + 1, 1 - b)
                pltpu.async_copy(ones_v, deg_sp.at[didx[b]],
                                 ssems[b], add=True)
            return 0
        lax.fori_loop(0, nchunks // 2, group, 0)
        wait_scatter((nchunks - 1) % 2)
        plsc.subcore_barrier()

        pltpu.sync_copy(deg_sp.at[pl.ds(row0, rows_per_tile)],
                        degp_hbm.at[cid, pl.ds(row0, rows_per_tile)])

    return deg_kernel(dst, zeros128)


def _sc_agg_call(z, src, dst):
    n, h = z.shape
    e = src.shape[0]
    epw = e // _NW
    nchunks = epw // _CHUNK
    assert epw * _NW == e and nchunks * _CHUNK == epw
    assert nchunks % _NBUF == 0
    np_ = _n_pad(n)
    rows_per_tile = np_ // _NS

    mesh = plsc.VectorSubcoreMesh(core_axis_name="c", subcore_axis_name="s")

    scratch = (
        pltpu.VMEM_SHARED((np_, h), jnp.float32),
        tuple(pltpu.VMEM((_CHUNK,), jnp.int32) for _ in range(_NBUF)),
        tuple(pltpu.VMEM((_CHUNK,), jnp.int32) for _ in range(_NBUF)),
        tuple(pltpu.VMEM((_CHUNK, h), jnp.float32) for _ in range(_NBUF)),
        tuple(pltpu.SemaphoreType.DMA for _ in range(_NBUF)),
        tuple(pltpu.SemaphoreType.DMA for _ in range(_NBUF)),
    )

    @functools.partial(
        pl.kernel,
        out_type=jax.ShapeDtypeStruct((_NC, np_, h), jnp.float32),
        mesh=mesh, scratch_types=scratch)
    def agg_kernel(z_hbm, src_hbm, dst_hbm, out_hbm,
                   agg_sp, sidx, didx, rows, gsems, ssems):
        cid = lax.axis_index("c")
        sid = lax.axis_index("s")

        z16 = jnp.zeros((16,), jnp.float32)
        lanes = h // 16

        def fill_zeros(i, _):
            rows[0][i // lanes, pl.ds((i % lanes) * 16, 16)] = z16
            return 0
        lax.fori_loop(0, _CHUNK * lanes, fill_zeros, 0)

        row0 = sid * rows_per_tile
        off = 0
        for sz in _chunk_sizes(rows_per_tile, _CHUNK):
            pltpu.sync_copy(rows[0].at[pl.ds(0, sz)],
                            agg_sp.at[pl.ds(row0 + off, sz)])
            off += sz
        plsc.subcore_barrier()

        base = (cid * _NS + sid) * epw

        def load_and_gather(i, b):
            off = base + i * _CHUNK
            pltpu.sync_copy(src_hbm.at[pl.ds(off, _CHUNK)], sidx[b])
            pltpu.sync_copy(dst_hbm.at[pl.ds(off, _CHUNK)], didx[b])
            pltpu.async_copy(z_hbm.at[sidx[b]], rows[b], gsems[b])

        def wait_gather(b):
            pltpu.make_async_copy(z_hbm.at[sidx[b]], rows[b],
                                  gsems[b]).wait()

        def wait_scatter(b):
            pltpu.make_async_copy(z_hbm.at[sidx[b]], rows[b],
                                  ssems[b]).wait()

        for b in range(_NBUF - 1):
            load_and_gather(b, b)

        def group(g, _):
            for b in range(_NBUF):
                i = g * _NBUF + b
                bp = (b - 1) % _NBUF
                wait_gather(b)
                pltpu.async_copy(rows[b], agg_sp.at[didx[b]],
                                 ssems[b], add=True)

                @pl.when(i >= 1)
                def _(bp=bp):
                    wait_scatter(bp)

                @pl.when(i + _NBUF - 1 < nchunks)
                def _(i=i, bp=bp):
                    load_and_gather(i + _NBUF - 1, bp)
            return 0
        lax.fori_loop(0, nchunks // _NBUF, group, 0)
        wait_scatter((nchunks - 1) % _NBUF)
        plsc.subcore_barrier()

        pltpu.sync_copy(agg_sp.at[pl.ds(row0, rows_per_tile)],
                        out_hbm.at[cid, pl.ds(row0, rows_per_tile)])

    return agg_kernel(z, src, dst)


def _proj_body(x_ref, wl_ref, wr_ref, bl_ref, z_ref, r_ref):
    xb = x_ref[...]
    z_ref[...] = jnp.dot(xb, wl_ref[...], preferred_element_type=jnp.float32)
    r_ref[...] = (jnp.dot(xb, wr_ref[...], preferred_element_type=jnp.float32)
                  + bl_ref[...])


def _tc_proj(x, wl, wr, bl):
    n, d = x.shape
    h = wl.shape[1]
    return pl.pallas_call(
        _proj_body,
        grid=(n // _BLK,),
        in_specs=[
            pl.BlockSpec((_BLK, d), lambda i: (i, 0)),
            pl.BlockSpec((d, h), lambda i: (0, 0)),
            pl.BlockSpec((d, h), lambda i: (0, 0)),
            pl.BlockSpec((1, h), lambda i: (0, 0)),
        ],
        out_specs=[pl.BlockSpec((_BLK, h), lambda i: (i, 0))] * 2,
        out_shape=[jax.ShapeDtypeStruct((n, h), jnp.float32)] * 2,
    )(x, wl, wr, bl)


def _make_combine_body(emit_out, project):
    def body(s_ref, dp_ref, r_ref, *rest):
        if project:
            wl_ref, wr_ref, bl_ref = rest[:3]
            rest = rest[3:]
        s = s_ref[0] + s_ref[1]
        deg = dp_ref[0] + dp_ref[1]
        inv = 1.0 / jnp.maximum(deg, 1.0)
        out = s * inv + r_ref[...]
        if project:
            hid = jnp.maximum(out, 0.0)
            if emit_out:
                out_ref, g_ref, z_ref, rn_ref = rest
                out_ref[...] = out
                g_ref[...] = hid
            else:
                z_ref, rn_ref = rest
            z_ref[...] = jnp.dot(hid, wl_ref[...],
                                 preferred_element_type=jnp.float32)
            rn_ref[...] = (jnp.dot(hid, wr_ref[...],
                                   preferred_element_type=jnp.float32)
                           + bl_ref[...])
        else:
            (xf_ref,) = rest
            xf_ref[...] = out
    return body


def _tc_combine(s, degp, r, wl=None, wr=None, bl=None, emit_out=False):
    n, h = r.shape
    project = wl is not None
    in_specs = [
        pl.BlockSpec((_NC, _BLK, h), lambda i: (0, i, 0)),
        pl.BlockSpec((_NC, _BLK, _DEGW), lambda i: (0, i, 0)),
        pl.BlockSpec((_BLK, h), lambda i: (i, 0)),
    ]
    args = [s, degp, r]
    n_out = 1
    if project:
        hn = wl.shape[1]
        in_specs += [
            pl.BlockSpec((h, hn), lambda i: (0, 0)),
            pl.BlockSpec((h, hn), lambda i: (0, 0)),
            pl.BlockSpec((1, hn), lambda i: (0, 0)),
        ]
        args += [wl, wr, bl]
        n_out = 4 if emit_out else 2
    outs = pl.pallas_call(
        _make_combine_body(emit_out, project),
        grid=(n // _BLK,),
        in_specs=in_specs,
        out_specs=[pl.BlockSpec((_BLK, h), lambda i: (i, 0))] * n_out,
        out_shape=[jax.ShapeDtypeStruct((n, h), jnp.float32)] * n_out,
    )(*args)
    return outs if n_out > 1 else outs[0]


def kernel(x, edge_index, Wl0, bl0, Wr0, Wl1, bl1, Wr1, Wl2, bl2, Wr2):
    n, d = x.shape
    h = Wl0.shape[1]
    c = Wl2.shape[1]
    e = edge_index.shape[1]

    epw = e // _NW
    assert epw * _NW == e
    step = _NBUF * _CHUNK
    epw_p = -(-epw // step) * step
    assert n < _n_pad(n)
    src = jnp.pad(edge_index[0].reshape(_NW, epw),
                  ((0, 0), (0, epw_p - epw))).reshape(-1)
    dst = jnp.pad(edge_index[1].reshape(_NW, epw),
                  ((0, 0), (0, epw_p - epw)),
                  constant_values=n).reshape(-1)

    bl0r = bl0.reshape(1, h)
    bl1r = bl1.reshape(1, h)
    Wl2p = jnp.pad(Wl2, ((0, 0), (0, h - c)))
    Wr2p = jnp.pad(Wr2, ((0, 0), (0, h - c)))
    bl2p = jnp.pad(bl2, (0, h - c)).reshape(1, h)

    degp = _sc_deg_call(dst, jnp.zeros((128, _DEGW), jnp.float32), n)
    z0, r0 = _tc_proj(x, Wl0, Wr0, bl0r)
    s0 = _sc_agg_call(z0, src, dst)
    z1, r1 = _tc_combine(s0, degp, r0, Wl1, Wr1, bl1r)
    s1 = _sc_agg_call(z1, src, dst)
    out1, g, z2, r2 = _tc_combine(s1, degp, r1, Wl2p, Wr2p, bl2p,
                                  emit_out=True)
    s2 = _sc_agg_call(z2, src, dst)
    xf_pad = _tc_combine(s2, degp, r2)
    return (xf_pad[:, :c], out1, g)


# speedup vs baseline: 2.4338x; 1.0533x over previous
import functools

import jax
import jax.numpy as jnp
from jax import lax
from jax.experimental import pallas as pl
from jax.experimental.pallas import tpu as pltpu
from jax.experimental.pallas import tpu_sc as plsc

_NC = 2
_NS = 16
_NW = _NC * _NS

_CHUNK = 120
_NBUF = 3
_DEGW = 128
_BLK = 1000


def _chunk_sizes(total, step):
    sizes = [step] * (total // step)
    if total % step:
        sizes.append(total % step)
    return sizes


def _n_pad(n):
    return -(-n // (_NS * 8)) * (_NS * 8)


def _sc_deg_call(dst, zeros128, n):
    e = dst.shape[0]
    epw = e // _NW
    nchunks = epw // _CHUNK
    assert epw * _NW == e and nchunks * _CHUNK == epw
    np_ = _n_pad(n)
    rows_per_tile = np_ // _NS

    mesh = plsc.VectorSubcoreMesh(core_axis_name="c", subcore_axis_name="s")

    @functools.partial(
        pl.kernel,
        out_type=jax.ShapeDtypeStruct((_NC, np_, _DEGW), jnp.float32),
        mesh=mesh,
        scratch_types=(
            pltpu.VMEM_SHARED((np_, _DEGW), jnp.float32),
            tuple(pltpu.VMEM((_CHUNK,), jnp.int32) for _ in range(2)),
            pltpu.VMEM((_CHUNK, _DEGW), jnp.float32),
            tuple(pltpu.SemaphoreType.DMA for _ in range(2)),
            tuple(pltpu.SemaphoreType.DMA for _ in range(2)),
        ))
    def deg_kernel(dst_hbm, zeros_hbm, degp_hbm, deg_sp, didx, ones_v,
                   isems, ssems):
        cid = lax.axis_index("c")
        sid = lax.axis_index("s")

        o16 = jnp.ones((16,), jnp.float32)
        lanes = _DEGW // 16

        def fill_ones(i, _):
            ones_v[i // lanes, pl.ds((i % lanes) * 16, 16)] = o16
            return 0
        lax.fori_loop(0, _CHUNK * lanes, fill_ones, 0)

        row0 = sid * rows_per_tile
        off = 0
        for sz in _chunk_sizes(rows_per_tile, zeros128.shape[0]):
            pltpu.sync_copy(zeros_hbm.at[pl.ds(0, sz)],
                            deg_sp.at[pl.ds(row0 + off, sz)])
            off += sz
        plsc.subcore_barrier()

        base = (cid * _NS + sid) * epw

        def load_idx(i, b):
            pltpu.async_copy(dst_hbm.at[pl.ds(base + i * _CHUNK, _CHUNK)],
                             didx[b], isems[b])

        def wait_idx(b):
            pltpu.make_async_copy(dst_hbm.at[pl.ds(base, _CHUNK)],
                                  didx[b], isems[b]).wait()

        def wait_scatter(b):
            pltpu.make_async_copy(ones_v, deg_sp.at[didx[b]],
                                  ssems[b]).wait()

        load_idx(0, 0)

        def group(g, _):
            for b in range(2):
                i = g * 2 + b
                wait_idx(b)

                @pl.when(i >= 1)
                def _(b=b):
                    wait_scatter(1 - b)

                @pl.when(i + 1 < nchunks)
                def _(i=i, b=b):
                    load_idx(i + 1, 1 - b)
                pltpu.async_copy(ones_v, deg_sp.at[didx[b]],
                                 ssems[b], add=True)
            return 0
        lax.fori_loop(0, nchunks // 2, group, 0)
        wait_scatter((nchunks - 1) % 2)
        plsc.subcore_barrier()

        pltpu.sync_copy(deg_sp.at[pl.ds(row0, rows_per_tile)],
                        degp_hbm.at[cid, pl.ds(row0, rows_per_tile)])

    return deg_kernel(dst, zeros128)


def _sc_agg_call(z, eidx):
    n, h = z.shape
    ncht = eidx.shape[0]
    nchunks = ncht // _NW
    assert nchunks * _NW == ncht
    assert nchunks % _NBUF == 0
    np_ = _n_pad(n)
    rows_per_tile = np_ // _NS

    mesh = plsc.VectorSubcoreMesh(core_axis_name="c", subcore_axis_name="s")

    scratch = (
        pltpu.VMEM_SHARED((np_, h), jnp.float32),
        tuple(pltpu.VMEM((2, _CHUNK), jnp.int32) for _ in range(_NBUF)),
        tuple(pltpu.VMEM((_CHUNK, h), jnp.float32) for _ in range(_NBUF)),
        tuple(pltpu.SemaphoreType.DMA for _ in range(_NBUF)),
        tuple(pltpu.SemaphoreType.DMA for _ in range(_NBUF)),
    )

    @functools.partial(
        pl.kernel,
        out_type=jax.ShapeDtypeStruct((_NC, np_, h), jnp.float32),
        mesh=mesh, scratch_types=scratch)
    def agg_kernel(z_hbm, eidx_hbm, out_hbm,
                   agg_sp, idxb, rows, gsems, ssems):
        cid = lax.axis_index("c")
        sid = lax.axis_index("s")

        z16 = jnp.zeros((16,), jnp.float32)
        lanes = h // 16

        def fill_zeros(i, _):
            rows[0][i // lanes, pl.ds((i % lanes) * 16, 16)] = z16
            return 0
        lax.fori_loop(0, _CHUNK * lanes, fill_zeros, 0)

        row0 = sid * rows_per_tile
        off = 0
        for sz in _chunk_sizes(rows_per_tile, _CHUNK):
            pltpu.sync_copy(rows[0].at[pl.ds(0, sz)],
                            agg_sp.at[pl.ds(row0 + off, sz)])
            off += sz
        plsc.subcore_barrier()

        base = (cid * _NS + sid) * nchunks

        def load_and_gather(i, b):
            pltpu.sync_copy(eidx_hbm.at[base + i], idxb[b])
            pltpu.async_copy(z_hbm.at[idxb[b].at[0]], rows[b], gsems[b])

        def wait_gather(b):
            pltpu.make_async_copy(z_hbm.at[idxb[b].at[0]], rows[b],
                                  gsems[b]).wait()

        def wait_scatter(b):
            pltpu.make_async_copy(z_hbm.at[idxb[b].at[0]], rows[b],
                                  ssems[b]).wait()

        for b in range(_NBUF - 1):
            load_and_gather(b, b)

        def group(g, _):
            for b in range(_NBUF):
                i = g * _NBUF + b
                bp = (b - 1) % _NBUF
                wait_gather(b)
                pltpu.async_copy(rows[b], agg_sp.at[idxb[b].at[1]],
                                 ssems[b], add=True)

                @pl.when(i >= 1)
                def _(bp=bp):
                    wait_scatter(bp)

                @pl.when(i + _NBUF - 1 < nchunks)
                def _(i=i, bp=bp):
                    load_and_gather(i + _NBUF - 1, bp)
            return 0
        lax.fori_loop(0, nchunks // _NBUF, group, 0)
        wait_scatter((nchunks - 1) % _NBUF)
        plsc.subcore_barrier()

        pltpu.sync_copy(agg_sp.at[pl.ds(row0, rows_per_tile)],
                        out_hbm.at[cid, pl.ds(row0, rows_per_tile)])

    return agg_kernel(z, eidx)


def _proj_body(x_ref, wl_ref, wr_ref, bl_ref, z_ref, r_ref):
    xb = x_ref[...]
    z_ref[...] = jnp.dot(xb, wl_ref[...], preferred_element_type=jnp.float32)
    r_ref[...] = (jnp.dot(xb, wr_ref[...], preferred_element_type=jnp.float32)
                  + bl_ref[...])


def _tc_proj(x, wl, wr, bl):
    n, d = x.shape
    h = wl.shape[1]
    return pl.pallas_call(
        _proj_body,
        grid=(n // _BLK,),
        in_specs=[
            pl.BlockSpec((_BLK, d), lambda i: (i, 0)),
            pl.BlockSpec((d, h), lambda i: (0, 0)),
            pl.BlockSpec((d, h), lambda i: (0, 0)),
            pl.BlockSpec((1, h), lambda i: (0, 0)),
        ],
        out_specs=[pl.BlockSpec((_BLK, h), lambda i: (i, 0))] * 2,
        out_shape=[jax.ShapeDtypeStruct((n, h), jnp.float32)] * 2,
    )(x, wl, wr, bl)


def _make_combine_body(emit_out, project):
    def body(s_ref, dp_ref, r_ref, *rest):
        if project:
            wl_ref, wr_ref, bl_ref = rest[:3]
            rest = rest[3:]
        s = s_ref[0] + s_ref[1]
        deg = dp_ref[0] + dp_ref[1]
        inv = 1.0 / jnp.maximum(deg, 1.0)
        out = s * inv + r_ref[...]
        if project:
            hid = jnp.maximum(out, 0.0)
            if emit_out:
                out_ref, g_ref, z_ref, rn_ref = rest
                out_ref[...] = out
                g_ref[...] = hid
            else:
                z_ref, rn_ref = rest
            z_ref[...] = jnp.dot(hid, wl_ref[...],
                                 preferred_element_type=jnp.float32)
            rn_ref[...] = (jnp.dot(hid, wr_ref[...],
                                   preferred_element_type=jnp.float32)
                           + bl_ref[...])
        else:
            (xf_ref,) = rest
            xf_ref[...] = out
    return body


def _tc_combine(s, degp, r, wl=None, wr=None, bl=None, emit_out=False):
    n, h = r.shape
    project = wl is not None
    in_specs = [
        pl.BlockSpec((_NC, _BLK, h), lambda i: (0, i, 0)),
        pl.BlockSpec((_NC, _BLK, _DEGW), lambda i: (0, i, 0)),
        pl.BlockSpec((_BLK, h), lambda i: (i, 0)),
    ]
    args = [s, degp, r]
    n_out = 1
    if project:
        hn = wl.shape[1]
        in_specs += [
            pl.BlockSpec((h, hn), lambda i: (0, 0)),
            pl.BlockSpec((h, hn), lambda i: (0, 0)),
            pl.BlockSpec((1, hn), lambda i: (0, 0)),
        ]
        args += [wl, wr, bl]
        n_out = 4 if emit_out else 2
    outs = pl.pallas_call(
        _make_combine_body(emit_out, project),
        grid=(n // _BLK,),
        in_specs=in_specs,
        out_specs=[pl.BlockSpec((_BLK, h), lambda i: (i, 0))] * n_out,
        out_shape=[jax.ShapeDtypeStruct((n, h), jnp.float32)] * n_out,
    )(*args)
    return outs if n_out > 1 else outs[0]


def kernel(x, edge_index, Wl0, bl0, Wr0, Wl1, bl1, Wr1, Wl2, bl2, Wr2):
    n, d = x.shape
    h = Wl0.shape[1]
    c = Wl2.shape[1]
    e = edge_index.shape[1]

    epw = e // _NW
    assert epw * _NW == e
    step = _NBUF * _CHUNK
    epw_p = -(-epw // step) * step
    assert n < _n_pad(n)
    src = jnp.pad(edge_index[0].reshape(_NW, epw),
                  ((0, 0), (0, epw_p - epw))).reshape(-1)
    dst = jnp.pad(edge_index[1].reshape(_NW, epw),
                  ((0, 0), (0, epw_p - epw)),
                  constant_values=n).reshape(-1)

    nchw = epw_p // _CHUNK
    eidx = jnp.stack([src.reshape(_NW * nchw, _CHUNK),
                      dst.reshape(_NW * nchw, _CHUNK)], axis=1)

    bl0r = bl0.reshape(1, h)
    bl1r = bl1.reshape(1, h)
    Wl2p = jnp.pad(Wl2, ((0, 0), (0, h - c)))
    Wr2p = jnp.pad(Wr2, ((0, 0), (0, h - c)))
    bl2p = jnp.pad(bl2, (0, h - c)).reshape(1, h)

    degp = _sc_deg_call(dst, jnp.zeros((128, _DEGW), jnp.float32), n)
    z0, r0 = _tc_proj(x, Wl0, Wr0, bl0r)
    s0 = _sc_agg_call(z0, eidx)
    z1, r1 = _tc_combine(s0, degp, r0, Wl1, Wr1, bl1r)
    s1 = _sc_agg_call(z1, eidx)
    out1, g, z2, r2 = _tc_combine(s1, degp, r1, Wl2p, Wr2p, bl2p,
                                  emit_out=True)
    s2 = _sc_agg_call(z2, eidx)
    xf_pad = _tc_combine(s2, degp, r2)
    return (xf_pad[:, :c], out1, g)
